# Initial kernel scaffold; baseline (speedup 1.0000x reference)
import jax
import jax.numpy as jnp
from jax.experimental import pallas as pl


def kernel(x, table):
    raise NotImplementedError("write your pallas kernel here")


# speedup vs baseline: 177.5595x; 177.5595x over previous
import functools

import jax
import jax.numpy as jnp
from jax import lax
from jax.experimental import pallas as pl
from jax.experimental.pallas import tpu as pltpu
from jax.experimental.pallas import tpu_sc as plsc

L = 16
NC = 2
NS = 16
NW = NC * NS
TOTAL = 16384 * 200
PER_W = TOTAL // NW
VECS = PER_W // L
UNROLL = 8

_mesh = plsc.VectorSubcoreMesh(core_axis_name="c", subcore_axis_name="s")


@functools.partial(
    pl.kernel,
    mesh=_mesh,
    compiler_params=pltpu.CompilerParams(needs_layout_passes=False),
    out_type=jax.ShapeDtypeStruct((NW, L), jnp.float32),
    scratch_types=[
        pltpu.VMEM((PER_W,), jnp.int32),
        pltpu.VMEM((32,), jnp.float32),
        pltpu.VMEM((5 * L,), jnp.float32),
        pltpu.VMEM((L,), jnp.float32),
    ],
)
def _lookup_sum(x_hbm, tflat_hbm, out_hbm, xbuf, tbuf, tab, accbuf):
    cid = lax.axis_index("c")
    sid = lax.axis_index("s")
    wid = sid * NC + cid
    base = wid * PER_W

    pltpu.sync_copy(x_hbm.at[pl.ds(base, PER_W)], xbuf)
    pltpu.sync_copy(tflat_hbm, tbuf)

    for k in range(5):
        v = tbuf[pl.ds(4 * k, L)]
        rs = v[0] + v[1] + v[2] + v[3]
        tab[pl.ds(k * L, L)] = jnp.broadcast_to(rs, (L,))

    lanes = lax.iota(jnp.int32, L)

    def body(i, acc):
        for u in range(UNROLL):
            v = xbuf[pl.ds((i * UNROLL + u) * L, L)]
            acc = acc + plsc.load_gather(tab, [v * L + lanes])
        return acc

    acc = lax.fori_loop(0, VECS // UNROLL, body, jnp.zeros((L,), jnp.float32))
    accbuf[...] = acc
    pltpu.sync_copy(accbuf, out_hbm.at[wid])


def kernel(x, table):
    xf = x.reshape(-1).astype(jnp.int32)
    tflat = jnp.zeros((32,), jnp.float32).at[:20].set(table.reshape(-1))
    partials = _lookup_sum(xf, tflat)
    return partials.sum()


# speedup vs baseline: 268.3064x; 1.5111x over previous
import functools

import jax
import jax.numpy as jnp
from jax import lax
from jax.experimental import pallas as pl
from jax.experimental.pallas import tpu as pltpu
from jax.experimental.pallas import tpu_sc as plsc

L = 16
NC = 2
NS = 16
NW = NC * NS
ROWS, COLS = 16384, 200
ROWS_W = ROWS // NW
NCH = 8
CHUNK = ROWS_W // NCH
FULL_VECS = COLS // L
TAIL_OFF = COLS - L
TAIL_DUP = FULL_VECS * L - TAIL_OFF

_mesh = plsc.VectorSubcoreMesh(core_axis_name="c", subcore_axis_name="s")


@functools.partial(
    pl.kernel,
    mesh=_mesh,
    compiler_params=pltpu.CompilerParams(needs_layout_passes=False),
    out_type=jax.ShapeDtypeStruct((NW, L), jnp.float32),
    scratch_types=[
        pltpu.VMEM((2, CHUNK, COLS), jnp.int32),
        pltpu.VMEM((32,), jnp.float32),
        pltpu.VMEM((5 * L,), jnp.float32),
        pltpu.VMEM((L,), jnp.float32),
        pltpu.SemaphoreType.DMA,
        pltpu.SemaphoreType.DMA,
    ],
)
def _lookup_sum(x_hbm, tflat_hbm, out_hbm, xbuf, tbuf, tab, accbuf, sem0, sem1):
    cid = lax.axis_index("c")
    sid = lax.axis_index("s")
    wid = sid * NC + cid
    row0 = wid * ROWS_W
    sems = (sem0, sem1)

    def chunk_copy(c, b):
        return pltpu.make_async_copy(
            x_hbm.at[pl.ds(row0 + c * CHUNK, CHUNK)], xbuf.at[b], sems[b]
        )

    chunk_copy(0, 0).start()
    pltpu.sync_copy(tflat_hbm, tbuf)

    for k in range(5):
        v = tbuf[pl.ds(4 * k, L)]
        rs = v[0] + v[1] + v[2] + v[3]
        tab[pl.ds(k * L, L)] = jnp.broadcast_to(rs, (L,))

    lanes = lax.iota(jnp.int32, L)
    tail_keep = lanes >= TAIL_DUP

    def make_body(buf):
        def body(r, accs):
            acc0, acc1 = accs
            for u in range(FULL_VECS):
                v = buf[r, pl.ds(u * L, L)]
                g = plsc.load_gather(tab, [v * L + lanes])
                if u % 2 == 0:
                    acc0 = acc0 + g
                else:
                    acc1 = acc1 + g
            v = buf[r, pl.ds(TAIL_OFF, L)]
            g = plsc.load_gather(tab, [v * L + lanes])
            acc0 = acc0 + jnp.where(tail_keep, g, 0.0)
            return acc0, acc1

        return body

    zeros = jnp.zeros((L,), jnp.float32)
    accs = (zeros, zeros)
    for c in range(NCH):
        b = c % 2
        chunk_copy(c, b).wait()
        if c + 1 < NCH:
            chunk_copy(c + 1, 1 - b).start()
        accs = lax.fori_loop(0, CHUNK, make_body(xbuf.at[b]), accs)
    accbuf[...] = accs[0] + accs[1]
    pltpu.sync_copy(accbuf, out_hbm.at[wid])


def kernel(x, table):
    tflat = jnp.zeros((32,), jnp.float32).at[:20].set(table.reshape(-1))
    partials = _lookup_sum(x, tflat)
    return partials.sum()


# speedup vs baseline: 331.0853x; 1.2340x over previous
import functools

import jax
import jax.numpy as jnp
from jax import lax
from jax.experimental import pallas as pl
from jax.experimental.pallas import tpu as pltpu
from jax.experimental.pallas import tpu_sc as plsc

L = 16
NC = 2
NS = 16
NW = NC * NS
ROWS, COLS = 16384, 200
ROWS_W = ROWS // NW
NCH = 4
CHUNK = ROWS_W // NCH
WORDS = COLS // 4
TAIL_WOFF = WORDS - L
TAIL_DUP = 3 * L - TAIL_WOFF
DUP_BYTES = 4 * TAIL_DUP
PAIR_STRIDE = 256
TAB_SIZE = ((4 + PAIR_STRIDE * 4) + 1) * L

_mesh = plsc.VectorSubcoreMesh(core_axis_name="c", subcore_axis_name="s")


@functools.partial(
    pl.kernel,
    mesh=_mesh,
    compiler_params=pltpu.CompilerParams(needs_layout_passes=False),
    out_type=jax.ShapeDtypeStruct((NW, L), jnp.float32),
    scratch_types=[
        pltpu.VMEM((2, CHUNK, COLS), jnp.int8),
        pltpu.VMEM((32,), jnp.float32),
        pltpu.VMEM((TAB_SIZE,), jnp.float32),
        pltpu.VMEM((L,), jnp.float32),
        pltpu.SemaphoreType.DMA,
        pltpu.SemaphoreType.DMA,
    ],
)
def _lookup_sum(x_hbm, tflat_hbm, out_hbm, xbuf, tbuf, tab, accbuf, sem0, sem1):
    cid = lax.axis_index("c")
    sid = lax.axis_index("s")
    wid = sid * NC + cid
    row0 = wid * ROWS_W
    sems = (sem0, sem1)

    def chunk_copy(c, b):
        return pltpu.make_async_copy(
            x_hbm.at[pl.ds(row0 + c * CHUNK, CHUNK)], xbuf.at[b], sems[b]
        )

    chunk_copy(0, 0).start()
    pltpu.sync_copy(tflat_hbm, tbuf)

    rs = []
    for k in range(5):
        v = tbuf[pl.ds(4 * k, L)]
        rs.append(v[0] + v[1] + v[2] + v[3])

    for b1 in range(5):
        for b0 in range(5):
            val = rs[b0] + rs[b1]
            tab[pl.ds((b0 + PAIR_STRIDE * b1) * L, L)] = jnp.broadcast_to(val, (L,))

    lanes = lax.iota(jnp.int32, L)
    tail_keep = lanes >= TAIL_DUP
    zero = jnp.zeros((L,), jnp.int32)

    def make_body(buf):
        def one_vec(r, woff, accs, tail):
            a0, a1 = accs
            bv = buf[r, pl.ds(woff * 4, 4 * L)]
            v = plsc.bitcast(bv, jnp.int32)
            if tail:
                v = jnp.where(tail_keep, v, zero)
            p0 = ((v & 0xFFFF) << 4) + lanes
            p1 = ((v >> 16) << 4) + lanes
            a0 = a0 + plsc.load_gather(tab, [p0])
            a1 = a1 + plsc.load_gather(tab, [p1])
            return a0, a1

        def body(r, accs):
            a01 = (accs[0], accs[1])
            a23 = (accs[2], accs[3])
            a01 = one_vec(r, 0, a01, False)
            a23 = one_vec(r, L, a23, False)
            a01 = one_vec(r, 2 * L, a01, False)
            a23 = one_vec(r, TAIL_WOFF, a23, True)
            return (a01[0], a01[1], a23[0], a23[1])

        return body

    zf = jnp.zeros((L,), jnp.float32)
    accs = (zf, zf, zf, zf)
    for c in range(NCH):
        b = c % 2
        chunk_copy(c, b).wait()
        if c + 1 < NCH:
            chunk_copy(c + 1, 1 - b).start()
        accs = lax.fori_loop(0, CHUNK, make_body(xbuf.at[b]), accs)

    correction = (ROWS_W * DUP_BYTES / L) * rs[0]
    total = accs[0] + accs[1] + accs[2] + accs[3]
    accbuf[...] = total - jnp.broadcast_to(correction, (L,))
    pltpu.sync_copy(accbuf, out_hbm.at[wid])


def kernel(x, table):
    x8 = x.astype(jnp.int8)
    tflat = jnp.zeros((32,), jnp.float32).at[:20].set(table.reshape(-1))
    partials = _lookup_sum(x8, tflat)
    return partials.sum()
